# Initial kernel scaffold; baseline (speedup 1.0000x reference)
import jax
import jax.numpy as jnp
from jax.experimental import pallas as pl


def kernel(x, W, bobot, edge_index):
    raise NotImplementedError("write your pallas kernel here")


# speedup vs baseline: 2.4629x; 2.4629x over previous
import functools

import jax
import jax.numpy as jnp
from jax import lax
from jax.experimental import pallas as pl
from jax.experimental.pallas import tpu as pltpu
from jax.experimental.pallas import tpu_sc as plsc

N = 10000
D = 128
E = 320000
MAX_ITER = 10
STEP = 0.1 / 1.5
YC = 16
NC = 2
NS = 16
NW = NC * NS
EPW = E // NW
B = 80
NBLK = EPW // B
NROWCHUNKS = N // B
_NCHUNK_CEIL = -(-NROWCHUNKS // NS)


def _sc_step_kernel():
    mesh = plsc.VectorSubcoreMesh(core_axis_name="c", subcore_axis_name="s",
                                  num_cores=NC, num_subcores=NS)

    def body(state_hbm, y_hbm, src_hbm, dst_hbm, w_hbm, out_hbm,
             acc, xs, xd, ys, yd, en, sidx, didx, wbuf, sem):
        cid = lax.axis_index("c")
        sid = lax.axis_index("s")
        wid = sid * NC + cid

        zero16 = jnp.zeros((16,), jnp.float32)

        def zbody(r, carry):
            for j in range(D // 16):
                xs[r, pl.ds(16 * j, 16)] = zero16
            return carry

        lax.fori_loop(0, B, zbody, 0)

        for k in range(_NCHUNK_CEIL):
            chunk = sid + k * NS

            @pl.when(chunk < NROWCHUNKS)
            def _():
                pltpu.sync_copy(xs, acc.at[pl.ds(chunk * B, B)])

        pltpu.sync_copy(w_hbm, wbuf)
        plsc.subcore_barrier()

        lanes = lax.iota(jnp.int32, 16)
        wk_vecs = [plsc.load_gather(wbuf, [jnp.full((16,), k, jnp.int32)])
                   for k in range(5)]

        def blk_body(blk, carry):
            base = wid * EPW + blk * B
            pltpu.sync_copy(src_hbm.at[pl.ds(base, B)], sidx)
            pltpu.sync_copy(dst_hbm.at[pl.ds(base, B)], didx)
            pltpu.async_copy(state_hbm.at[sidx], xs, sem).wait()
            pltpu.async_copy(state_hbm.at[didx], xd, sem).wait()
            pltpu.async_copy(y_hbm.at[sidx], ys, sem).wait()
            pltpu.async_copy(y_hbm.at[didx], yd, sem).wait()

            for g in range(B // 16):
                rows = g * 16 + lanes
                e_acc = jnp.zeros((16,), jnp.float32)
                for k in range(5):
                    col = jnp.full((16,), k, jnp.int32)
                    a = plsc.load_gather(ys, [rows, col])
                    b = plsc.load_gather(yd, [rows, col])
                    e_acc = e_acc + wk_vecs[k] * jnp.abs(a - b)
                en[pl.ds(g * 16, 16)] = e_acc

            def mbody(r, carry):
                s = plsc.load_gather(en, [jnp.full((16,), r, jnp.int32)])
                for j in range(D // 16):
                    a = xs[r, pl.ds(16 * j, 16)]
                    b = xd[r, pl.ds(16 * j, 16)]
                    m = (a - b) * s
                    xs[r, pl.ds(16 * j, 16)] = m
                    xd[r, pl.ds(16 * j, 16)] = -m
                return carry

            lax.fori_loop(0, B, mbody, 0)

            pltpu.sync_copy(xs, acc.at[didx], add=True)
            pltpu.sync_copy(xd, acc.at[sidx], add=True)
            return carry

        lax.fori_loop(0, NBLK, blk_body, 0)
        plsc.subcore_barrier()

        for k in range(_NCHUNK_CEIL):
            chunk = sid + k * NS

            @pl.when(chunk < NROWCHUNKS)
            def _():
                pltpu.sync_copy(acc.at[pl.ds(chunk * B, B)],
                                out_hbm.at[pl.ds(cid * N + chunk * B, B)])

    return pl.kernel(
        body,
        out_type=jax.ShapeDtypeStruct((2 * N, D), jnp.float32),
        mesh=mesh,
        scratch_types=[
            pltpu.VMEM_SHARED((N, D), jnp.float32),
            pltpu.VMEM((B, D), jnp.float32),
            pltpu.VMEM((B, D), jnp.float32),
            pltpu.VMEM((B, YC), jnp.float32),
            pltpu.VMEM((B, YC), jnp.float32),
            pltpu.VMEM((B,), jnp.float32),
            pltpu.VMEM((B,), jnp.int32),
            pltpu.VMEM((B,), jnp.int32),
            pltpu.VMEM((16,), jnp.float32),
            pltpu.SemaphoreType.DMA,
        ],
        compiler_params=pltpu.CompilerParams(needs_layout_passes=False,
                                             use_tc_tiling_on_sc=False),
    )


RB = 1000


def _proj_body(s_ref, w_ref, y_ref):
    y_ref[...] = jnp.dot(s_ref[...], w_ref[...],
                         preferred_element_type=jnp.float32)


def _update_body(s_ref, a0_ref, a1_ref, w_ref, o_ref, y_ref):
    ns = s_ref[...] - STEP * (a0_ref[...] + a1_ref[...])
    o_ref[...] = ns
    y_ref[...] = jnp.dot(ns, w_ref[...], preferred_element_type=jnp.float32)


def _make_tc_kernels():
    grid = (N // RB,)
    s_spec = pl.BlockSpec((RB, D), lambda i: (i, 0))
    w_spec = pl.BlockSpec((D, YC), lambda i: (0, 0))
    y_spec = pl.BlockSpec((RB, YC), lambda i: (i, 0))
    proj = pl.pallas_call(
        _proj_body,
        grid=grid,
        in_specs=[s_spec, w_spec],
        out_specs=y_spec,
        out_shape=jax.ShapeDtypeStruct((N, YC), jnp.float32),
    )
    a0_spec = pl.BlockSpec((RB, D), lambda i: (i, 0))
    a1_spec = pl.BlockSpec((RB, D), lambda i: (i + N // RB, 0))
    update = pl.pallas_call(
        _update_body,
        grid=grid,
        in_specs=[s_spec, a0_spec, a1_spec, w_spec],
        out_specs=[s_spec, y_spec],
        out_shape=[jax.ShapeDtypeStruct((N, D), jnp.float32),
                   jax.ShapeDtypeStruct((N, YC), jnp.float32)],
    )
    return proj, update


def kernel(x, W, bobot, edge_index):
    w = jax.nn.softmax(bobot)
    w16 = jnp.zeros((16,), jnp.float32).at[:5].set(w)
    Wp = jnp.zeros((D, YC), jnp.float32).at[:, :5].set(W)
    src = edge_index[0]
    dst = edge_index[1]

    sc_step = _sc_step_kernel()
    proj, update = _make_tc_kernels()

    state = x
    Y = proj(state, Wp)
    for _ in range(MAX_ITER):
        acc = sc_step(state, Y, src, dst, w16)
        state, Y = update(state, acc, acc, Wp)
    return state


# speedup vs baseline: 3.0386x; 1.2338x over previous
import functools

import jax
import jax.numpy as jnp
from jax import lax
from jax.experimental import pallas as pl
from jax.experimental.pallas import tpu as pltpu
from jax.experimental.pallas import tpu_sc as plsc

N = 10000
D = 128
E = 320000
MAX_ITER = 10
STEP = 0.1 / 1.5
YC = 16
NC = 2
NS = 16
NW = NC * NS
EPW = E // NW
B = 80
NBLK = EPW // B
NROWCHUNKS = N // B
_NCHUNK_CEIL = -(-NROWCHUNKS // NS)


def _sc_step_kernel():
    mesh = plsc.VectorSubcoreMesh(core_axis_name="c", subcore_axis_name="s",
                                  num_cores=NC, num_subcores=NS)

    def body(state_hbm, y_hbm, src_hbm, dst_hbm, w_hbm, out_hbm,
             acc, xs, xd, ys, yd, en, sidx, didx, wbuf, gsem, ssem):
        cid = lax.axis_index("c")
        sid = lax.axis_index("s")
        wid = sid * NC + cid

        zero16 = jnp.zeros((16,), jnp.float32)

        def zbody(r, carry):
            for j in range(D // 16):
                xs[r, pl.ds(16 * j, 16)] = zero16
            return carry

        lax.fori_loop(0, B, zbody, 0)

        for k in range(_NCHUNK_CEIL):
            chunk = sid + k * NS

            @pl.when(chunk < NROWCHUNKS)
            def _():
                pltpu.sync_copy(xs, acc.at[pl.ds(chunk * B, B)])

        pltpu.sync_copy(w_hbm, wbuf)
        plsc.subcore_barrier()

        lanes = lax.iota(jnp.int32, 16)
        wk_vecs = [plsc.load_gather(wbuf, [jnp.full((16,), k, jnp.int32)])
                   for k in range(5)]

        def blk_body(blk, carry):
            base = wid * EPW + blk * B

            @pl.when(blk > 0)
            def _():
                pltpu.make_async_copy(xs, acc.at[didx], ssem).wait()
                pltpu.make_async_copy(xd, acc.at[sidx], ssem).wait()

            pltpu.sync_copy(src_hbm.at[pl.ds(base, B)], sidx)
            pltpu.sync_copy(dst_hbm.at[pl.ds(base, B)], didx)
            cps = [pltpu.async_copy(state_hbm.at[sidx], xs, gsem),
                   pltpu.async_copy(state_hbm.at[didx], xd, gsem),
                   pltpu.async_copy(y_hbm.at[sidx], ys, gsem),
                   pltpu.async_copy(y_hbm.at[didx], yd, gsem)]
            for cp in cps:
                cp.wait()

            for g in range(B // 16):
                rows = g * 16 + lanes
                e_acc = jnp.zeros((16,), jnp.float32)
                for k in range(5):
                    col = jnp.full((16,), k, jnp.int32)
                    a = plsc.load_gather(ys, [rows, col])
                    b = plsc.load_gather(yd, [rows, col])
                    e_acc = e_acc + wk_vecs[k] * jnp.abs(a - b)
                en[pl.ds(g * 16, 16)] = e_acc

            def mbody(r, carry):
                s = plsc.load_gather(en, [jnp.full((16,), r, jnp.int32)])
                for j in range(D // 16):
                    a = xs[r, pl.ds(16 * j, 16)]
                    b = xd[r, pl.ds(16 * j, 16)]
                    m = (a - b) * s
                    xs[r, pl.ds(16 * j, 16)] = m
                    xd[r, pl.ds(16 * j, 16)] = -m
                return carry

            lax.fori_loop(0, B, mbody, 0)

            pltpu.async_copy(xs, acc.at[didx], ssem, add=True)
            pltpu.async_copy(xd, acc.at[sidx], ssem, add=True)
            return carry

        lax.fori_loop(0, NBLK, blk_body, 0)
        pltpu.make_async_copy(xs, acc.at[didx], ssem).wait()
        pltpu.make_async_copy(xd, acc.at[sidx], ssem).wait()
        plsc.subcore_barrier()

        for k in range(_NCHUNK_CEIL):
            chunk = sid + k * NS

            @pl.when(chunk < NROWCHUNKS)
            def _():
                pltpu.sync_copy(acc.at[pl.ds(chunk * B, B)],
                                out_hbm.at[pl.ds(cid * N + chunk * B, B)])

    return pl.kernel(
        body,
        out_type=jax.ShapeDtypeStruct((2 * N, D), jnp.float32),
        mesh=mesh,
        scratch_types=[
            pltpu.VMEM_SHARED((N, D), jnp.float32),
            pltpu.VMEM((B, D), jnp.float32),
            pltpu.VMEM((B, D), jnp.float32),
            pltpu.VMEM((B, YC), jnp.float32),
            pltpu.VMEM((B, YC), jnp.float32),
            pltpu.VMEM((B,), jnp.float32),
            pltpu.VMEM((B,), jnp.int32),
            pltpu.VMEM((B,), jnp.int32),
            pltpu.VMEM((16,), jnp.float32),
            pltpu.SemaphoreType.DMA,
            pltpu.SemaphoreType.DMA,
        ],
        compiler_params=pltpu.CompilerParams(needs_layout_passes=False,
                                             use_tc_tiling_on_sc=False),
    )


RB = 1000


def _proj_body(s_ref, w_ref, y_ref):
    y_ref[...] = jnp.dot(s_ref[...], w_ref[...],
                         preferred_element_type=jnp.float32)


def _update_body(s_ref, a0_ref, a1_ref, w_ref, o_ref, y_ref):
    ns = s_ref[...] - STEP * (a0_ref[...] + a1_ref[...])
    o_ref[...] = ns
    y_ref[...] = jnp.dot(ns, w_ref[...], preferred_element_type=jnp.float32)


def _make_tc_kernels():
    grid = (N // RB,)
    s_spec = pl.BlockSpec((RB, D), lambda i: (i, 0))
    w_spec = pl.BlockSpec((D, YC), lambda i: (0, 0))
    y_spec = pl.BlockSpec((RB, YC), lambda i: (i, 0))
    proj = pl.pallas_call(
        _proj_body,
        grid=grid,
        in_specs=[s_spec, w_spec],
        out_specs=y_spec,
        out_shape=jax.ShapeDtypeStruct((N, YC), jnp.float32),
    )
    a0_spec = pl.BlockSpec((RB, D), lambda i: (i, 0))
    a1_spec = pl.BlockSpec((RB, D), lambda i: (i + N // RB, 0))
    update = pl.pallas_call(
        _update_body,
        grid=grid,
        in_specs=[s_spec, a0_spec, a1_spec, w_spec],
        out_specs=[s_spec, y_spec],
        out_shape=[jax.ShapeDtypeStruct((N, D), jnp.float32),
                   jax.ShapeDtypeStruct((N, YC), jnp.float32)],
    )
    return proj, update


def kernel(x, W, bobot, edge_index):
    w = jax.nn.softmax(bobot)
    w16 = jnp.zeros((16,), jnp.float32).at[:5].set(w)
    Wp = jnp.zeros((D, YC), jnp.float32).at[:, :5].set(W)
    src = edge_index[0]
    dst = edge_index[1]

    sc_step = _sc_step_kernel()
    proj, update = _make_tc_kernels()

    state = x
    Y = proj(state, Wp)
    for _ in range(MAX_ITER):
        acc = sc_step(state, Y, src, dst, w16)
        state, Y = update(state, acc, acc, Wp)
    return state
